# baseline (speedup 1.0000x reference)
import functools
import math

import jax
import jax.numpy as jnp
import numpy as np
from jax.experimental import pallas as pl

N = 10000
K = 30
E = N * K
H = 32
NUM_AA = 20
L = 2


def _mm_body(nx, act, *refs):
    x_refs = refs[:nx]
    w_refs = refs[nx:2 * nx]
    rest = refs[2 * nx:]
    add_ref = rest[0] if len(rest) == 2 else None
    out_ref = rest[-1]
    acc = None
    for xr, wr in zip(x_refs, w_refs):
        p = jnp.dot(xr[...], wr[...], preferred_element_type=jnp.float32)
        acc = p if acc is None else acc + p
    if add_ref is not None:
        acc = acc + add_ref[...]
    if act == "silu":
        acc = acc * (1.0 / (1.0 + jnp.exp(-acc)))
    out_ref[...] = acc


def _mm(pairs, add=None, act="none", block_rows=1024):
    R = pairs[0][0].shape[0]
    Cout = pairs[0][1].shape[1]
    BR = min(block_rows, ((R + 7) // 8) * 8)
    Rp = ((R + BR - 1) // BR) * BR
    xs = []
    for X, W in pairs:
        if Rp != R:
            X = jnp.pad(X, ((0, Rp - R), (0, 0)))
        xs.append(X)
    ops = list(xs) + [W for _, W in pairs]
    in_specs = [pl.BlockSpec((BR, X.shape[1]), lambda i: (i, 0)) for X in xs]
    in_specs += [pl.BlockSpec(W.shape, lambda i: (0, 0)) for _, W in pairs]
    if add is not None:
        if Rp != R:
            add = jnp.pad(add, ((0, Rp - R), (0, 0)))
        ops.append(add)
        in_specs.append(pl.BlockSpec((BR, Cout), lambda i: (i, 0)))
    out = pl.pallas_call(
        functools.partial(_mm_body, len(pairs), act),
        grid=(Rp // BR,),
        in_specs=in_specs,
        out_specs=pl.BlockSpec((BR, Cout), lambda i: (i, 0)),
        out_shape=jax.ShapeDtypeStruct((Rp, Cout), jnp.float32),
    )(*ops)
    return out[:R] if Rp != R else out


def _normalize(v, eps=1e-8):
    return v / (jnp.linalg.norm(v, axis=-1, keepdims=True) + eps)


def _dihedrals(bb, eps=1e-7):
    X = bb[:, :3, :].reshape(-1, 3)
    dX = X[1:] - X[:-1]
    U = _normalize(dX)
    u2, u1, u0 = U[:-2], U[1:-1], U[2:]
    n2 = _normalize(jnp.cross(u2, u1))
    n1 = _normalize(jnp.cross(u1, u0))
    cosD = jnp.clip(jnp.sum(n2 * n1, -1), -1 + eps, 1 - eps)
    D = jnp.sign(jnp.sum(u2 * n1, -1)) * jnp.arccos(cosD)
    D = jnp.concatenate([jnp.zeros(1), D, jnp.zeros(2)])
    D = D.reshape(-1, 3)
    feats = jnp.concatenate([jnp.cos(D), jnp.sin(D)], -1)
    return jnp.pad(feats, ((0, 0), (0, 1)))


def _orientations(X_ca):
    f = _normalize(X_ca[1:] - X_ca[:-1])
    forward = jnp.concatenate([f, jnp.zeros((1, 3))], 0)
    backward = jnp.concatenate([jnp.zeros((1, 3)), -f], 0)
    return jnp.stack([forward, backward], axis=1)


def _virtual_cb(bb):
    X_ca = bb[:, 1, :]
    b = X_ca - bb[:, 0, :]
    c = bb[:, 2, :] - X_ca
    a = jnp.cross(b, c)
    return -0.58273431 * a + 0.56802827 * b - 0.54067466 * c


def _rbf(D, D_count=16, D_min=0.0, D_max=20.0):
    mu = jnp.linspace(D_min, D_max, D_count)
    sigma = (D_max - D_min) / D_count
    return jnp.exp(-(((D[..., None] - mu) / sigma) ** 2))


def _posemb(ei, num_embeddings=16):
    d = (ei[0] - ei[1]).astype(jnp.float32)
    freq = jnp.exp(jnp.arange(0, num_embeddings, 2, dtype=jnp.float32) * (-np.log(10000.0) / num_embeddings))
    ang = d[:, None] * freq[None, :]
    return jnp.concatenate([jnp.cos(ang), jnp.sin(ang)], axis=-1)


def _conv(x, ef, src, dst, rdeg, W0, We, W1):
    nI = x.shape[1]
    m0n = _mm([(x[:, 0, :], W0)])
    m1n = _mm([(x[:, 1:, :].reshape(N * (nI - 1), x.shape[2]), W1)])
    M = jnp.concatenate([m0n[:, None, :], m1n.reshape(N, nI - 1, H)], axis=1)
    efWe = _mm([(ef, We)])
    msg = M[src]
    msg = msg.at[:, 0, :].add(efWe)
    agg = jax.ops.segment_sum(msg, dst, num_segments=N)
    return agg * rdeg[:, None, None]


def _block(x, ef, src, dst, Wq, Wk, Wv, Wa, Wo, Wf1, Wf2):
    nI, C = x.shape[1], x.shape[2]
    Co = Wv.shape[1]
    q = _mm([(x[:, 0, :], Wq)])
    kk = _mm([(x[:, 0, :], Wk)])
    efWa = _mm([(ef, Wa)])[:, 0]
    logits = jnp.sum(q[dst] * kk[src], -1) / np.sqrt(Wq.shape[1]) + efWa
    m = jax.ops.segment_max(logits, dst, num_segments=N)
    ex = jnp.exp(logits - m[dst])
    den = jax.ops.segment_sum(ex, dst, num_segments=N)
    alpha = ex / (den[dst] + 1e-9)
    v = _mm([(x.reshape(N * nI, C), Wv)]).reshape(N, nI, Co)
    agg = jax.ops.segment_sum(alpha[:, None, None] * v[src], dst, num_segments=N)
    y = _mm([(agg.reshape(N * nI, Co), Wo)]).reshape(N, nI, Co)
    if Co == C:
        y = y + x
    u0 = _mm([(y[:, 0, :], Wf1)], act="silu")
    u1 = _mm([(y[:, 1:, :].reshape(N * (nI - 1), Co), Wf1)])
    f0 = _mm([(u0, Wf2)])
    f1 = _mm([(u1, Wf2)]).reshape(N, nI - 1, Co)
    return y + jnp.concatenate([f0[:, None, :], f1], axis=1)


def kernel(bb, seq, atom91_centered, edge_index, W0_bb, We_bb, W1_bb, W0_at, We_at, W1_at, Wq, Wk, Wv, Wa, Wo, Wf1, Wf2, Weu, Wq_mu, Wk_mu, Wv_mu, Wa_mu, Wo_mu, Wf1_mu, Wf2_mu, Wq_lv, Wk_lv, Wv_lv, Wa_lv, Wo_lv, Wf1_lv, Wf2_lv):
    src, dst = edge_index[0], edge_index[1]
    X_ca = bb[:, 1, :]
    cond_seq = jax.nn.one_hot(seq, NUM_AA, dtype=jnp.float32)
    dih = _dihedrals(bb)
    ori = _orientations(X_ca)
    vcb = _virtual_cb(bb)
    bb_rel = bb - X_ca[:, None, :]
    vecs = jnp.nan_to_num(jnp.concatenate([bb_rel, ori, vcb[:, None, :]], axis=1))
    bb_feat = jnp.concatenate([dih[:, None, :], jnp.swapaxes(vecs, 1, 2)], axis=1)
    atom = jnp.zeros((N, 4, 91), dtype=jnp.float32)
    atom = atom.at[:, 1:4, :].set(atom91_centered)
    atom = atom.at[:, :, :NUM_AA].set(jnp.broadcast_to(cond_seq[:, None, :], (N, 4, NUM_AA)))

    evec = X_ca[dst] - X_ca[src]
    edist = jnp.linalg.norm(evec, axis=-1)
    ef = jnp.concatenate([_rbf(edist), _posemb(edge_index)], axis=-1)

    deg = jax.ops.segment_sum(jnp.ones((E,), jnp.float32), dst, num_segments=N)
    rdeg = 1.0 / jnp.maximum(deg, 1.0)

    bb_h = _conv(bb_feat, ef, src, dst, rdeg, W0_bb, We_bb, W1_bb)
    at_h = _conv(atom, ef, src, dst, rdeg, W0_at, We_at, W1_at)
    x = jnp.concatenate([bb_h, at_h], axis=-1)

    for i in range(L):
        x = _block(x, ef, src, dst, Wq[i], Wk[i], Wv[i], Wa[i], Wo[i], Wf1[i], Wf2[i])
        x0 = x[:, 0, :]
        A = _mm([(x0, Weu[i][:2 * H])])
        B = _mm([(x0, Weu[i][2 * H:4 * H])])
        C = _mm([(ef, Weu[i][4 * H:])])
        z = A[src] + B[dst] + C
        ef = z * (1.0 / (1.0 + jnp.exp(-z)))

    mu = _block(x, ef, src, dst, Wq_mu, Wk_mu, Wv_mu, Wa_mu, Wo_mu, Wf1_mu, Wf2_mu)
    lv = _block(x, ef, src, dst, Wq_lv, Wk_lv, Wv_lv, Wa_lv, Wo_lv, Wf1_lv, Wf2_lv)
    return jnp.stack([mu, lv], axis=0)
